# Initial kernel scaffold; baseline (speedup 1.0000x reference)
import functools

import jax
import jax.numpy as jnp
from jax import lax
from jax.experimental import pallas as pl
from jax.experimental.pallas import tpu as pltpu
from jax.experimental.pallas import tpu_sc as plsc

CLS = 80
NQ = 1000
K = 300
N = NQ * CLS
NV = N // 16
NB = 1024
NBC = NB // 16
CAP = 8192
PAD = 320
BATCH = 64


def _scalar(x):
    return x[0] if getattr(x, "ndim", 0) else x


def _sc_body(scores_hbm, boxes_hbm, sizes_hbm, out_s_hbm, out_l_hbm, out_b_hbm,
             sc_v, bx_v, sz_v, hist_v, tot_v, cu_v, ci_v, fu_v, fi_v,
             uu_v, oi_v, ou_s, ol_v, ob_v):
    lane = lax.iota(jnp.int32, 16)
    ones = jnp.ones((16,), jnp.int32)
    zz = jnp.zeros((16,), jnp.int32)
    big = jnp.int32(1 << 30)
    wid = lax.axis_index("s") * 2 + lax.axis_index("c")
    pltpu.sync_copy(sizes_hbm, sz_v)

    def _zero_hist():
        def zb(i, _):
            hist_v[pl.ds(i * 16, 16)] = zz
            return 0
        lax.fori_loop(0, NB, zb, 0)

    def _rowsums():
        def rs_chunk(jc, _):
            acc = zz
            for l in range(16):
                row = hist_v[pl.ds((jc * 16 + l) * 16, 16)]
                acc = jnp.where(lane == l, jnp.sum(row), acc)
            tot_v[pl.ds(jc * 16, 16)] = acc
            return 0
        lax.fori_loop(0, NBC, rs_chunk, 0)

    def _boundary(tgt):
        def chunk(jj, carry):
            fb, fna, cum = carry
            jc = NBC - 1 - jj
            rs = tot_v[pl.ds(jc * 16, 16)]
            cs = plsc.cumsum(lax.rev(rs, (0,)))
            cond = (cum + cs) >= tgt
            t = _scalar(plsc.all_reduce_ffs(cond))
            hit = (t < 16) & (fb < 0)
            na_extra = jnp.sum(jnp.where(lane == t - 1, cs, 0))
            b_new = jc * 16 + 15 - t
            fb = jnp.where(hit, b_new, fb)
            fna = jnp.where(hit, cum + na_extra, fna)
            cum = cum + cs[15]
            return fb, fna, cum
        fb, fna, _ = lax.fori_loop(
            0, NBC, chunk, (jnp.int32(-1), jnp.int32(0), jnp.int32(0)))
        return fb, fna

    def per_image(ii, _):
        b = wid * 2 + ii
        pltpu.sync_copy(scores_hbm.at[pl.ds(b * N, N)], sc_v)
        pltpu.sync_copy(boxes_hbm.at[pl.ds(b * 4 * NQ, 4 * NQ)], bx_v)

        def load_full(i):
            v = sc_v[pl.ds(i * 16, 16)]
            u = jnp.maximum(plsc.bitcast(v, jnp.int32), 1)
            return u, i * 16 + lane, None

        def and_valid(cond, valid):
            return cond if valid is None else (cond & valid)

        _zero_hist()

        def h1(i, _):
            u, _idx, _val = load_full(i)
            d = u >> 20
            plsc.addupdate_scatter(hist_v, [d * 16 + lane], ones)
            return 0
        lax.fori_loop(0, NV, h1, 0)
        _rowsums()
        b1, n_hi1 = _boundary(jnp.int32(K))
        row = tot_v[pl.ds((b1 // 16) * 16, 16)]
        cnt_b1 = jnp.sum(jnp.where(lane == (b1 % 16), row, 0))
        nc = n_hi1 + cnt_b1

        def refine(load_fn, nloops):
            _zero_hist()

            def h2(i, _):
                u, _idx, valid = load_fn(i)
                m = and_valid((u >> 20) == b1, valid)
                d = (u >> 10) & 0x3FF
                plsc.addupdate_scatter(hist_v, [d * 16 + lane], ones, mask=m)
                return 0
            lax.fori_loop(0, nloops, h2, 0)
            _rowsums()
            b2, n_hi2 = _boundary(K - n_hi1)
            pre2 = (b1 << 10) | b2
            _zero_hist()

            def h3(i, _):
                u, _idx, valid = load_fn(i)
                m = and_valid((u >> 10) == pre2, valid)
                d = u & 0x3FF
                plsc.addupdate_scatter(hist_v, [d * 16 + lane], ones, mask=m)
                return 0
            lax.fori_loop(0, nloops, h3, 0)
            _rowsums()
            b3, n_hi3 = _boundary(K - n_hi1 - n_hi2)
            thr = (pre2 << 10) | b3
            n_strict = n_hi1 + n_hi2 + n_hi3
            m_eq = K - n_strict

            def cps(i, off):
                u, idx, valid = load_fn(i)
                m = and_valid(u > thr, valid)
                plsc.store_compressed(fu_v.at[pl.ds(off, 16)], u, mask=m)
                plsc.store_compressed(fi_v.at[pl.ds(off, 16)], idx, mask=m)
                return off + _scalar(plsc.all_reduce_population_count(m))
            lax.fori_loop(0, nloops, cps, jnp.int32(0))

            def cpe(i, off):
                u, idx, valid = load_fn(i)
                m = and_valid(u == thr, valid)
                dst = n_strict + jnp.minimum(off, m_eq)
                plsc.store_compressed(fu_v.at[pl.ds(dst, 16)], u, mask=m)
                plsc.store_compressed(fi_v.at[pl.ds(dst, 16)], idx, mask=m)
                return off + _scalar(plsc.all_reduce_population_count(m))
            lax.fori_loop(0, nloops, cpe, jnp.int32(0))
            fu_v[pl.ds(K, 16)] = zz

        @pl.when(nc <= CAP)
        def _():
            def cp(i, off):
                u, idx, _val = load_full(i)
                m = (u >> 20) >= b1
                plsc.store_compressed(cu_v.at[pl.ds(off, 16)], u, mask=m)
                plsc.store_compressed(ci_v.at[pl.ds(off, 16)], idx, mask=m)
                return off + _scalar(plsc.all_reduce_population_count(m))
            lax.fori_loop(0, NV, cp, jnp.int32(0))

            def load_c(i):
                u = cu_v[pl.ds(i * 16, 16)]
                idx = ci_v[pl.ds(i * 16, 16)]
                return u, idx, (i * 16 + lane) < nc
            refine(load_c, (nc + 15) >> 4)

        @pl.when(nc > CAP)
        def _():
            refine(load_full, NV)

        def ext(e, carry):
            curv, curi = carry
            m_run = zz
            j_run = zz
            for j in range(19):
                w = fu_v[pl.ds(j * 16, 16)]
                upd = w > m_run
                j_run = jnp.where(upd, j, j_run)
                m_run = jnp.where(upd, w, m_run)
            m = jnp.max(m_run)
            pv = jnp.where(m_run == m, j_run * 16 + lane, big)
            p = jnp.min(pv)
            fidxv = plsc.load_gather(fi_v, [zz + p])
            jc = p >> 4
            w2 = fu_v[pl.ds(jc * 16, 16)]
            fu_v[pl.ds(jc * 16, 16)] = jnp.where(lane == (p & 15), 0, w2)
            el = e & 15
            curv = jnp.where(lane == el, m, curv)
            curi = jnp.where(lane == el, fidxv, curi)

            @pl.when(el == 15)
            def _():
                eb = (e >> 4) * 16
                uu_v[pl.ds(eb, 16)] = curv
                oi_v[pl.ds(eb, 16)] = curi
            return curv, curi
        lax.fori_loop(0, PAD - 16, ext, (zz, zz))

        wv = plsc.load_gather(sz_v, [zz + b * 16])
        hv = plsc.load_gather(sz_v, [zz + b * 16 + 1])

        def post(j, _):
            ub = uu_v[pl.ds(j * 16, 16)]
            ou_s[pl.ds(j * 16, 16)] = plsc.bitcast(ub, jnp.float32)
            fidx = oi_v[pl.ds(j * 16, 16)]
            ol_v[pl.ds(j * 16, 16)] = fidx % CLS
            q4 = (fidx // CLS) * 4
            cx = plsc.load_gather(bx_v, [q4])
            cy = plsc.load_gather(bx_v, [q4 + 1])
            ww = plsc.load_gather(bx_v, [q4 + 2])
            hh = plsc.load_gather(bx_v, [q4 + 3])
            pos = (j * 16 + lane) * 4
            plsc.store_scatter(ob_v, [pos], (cx - 0.5 * ww) * wv)
            plsc.store_scatter(ob_v, [pos + 1], (cy - 0.5 * hh) * hv)
            plsc.store_scatter(ob_v, [pos + 2], (cx + 0.5 * ww) * wv)
            plsc.store_scatter(ob_v, [pos + 3], (cy + 0.5 * hh) * hv)
            return 0
        lax.fori_loop(0, PAD // 16, post, 0)

        pltpu.sync_copy(ou_s, out_s_hbm.at[pl.ds(b * PAD, PAD)])
        pltpu.sync_copy(ol_v, out_l_hbm.at[pl.ds(b * PAD, PAD)])
        pltpu.sync_copy(ob_v, out_b_hbm.at[pl.ds(b * PAD * 4, PAD * 4)])
        return 0

    lax.fori_loop(0, 2, per_image, 0)


@functools.cache
def _make_sc_call():
    return pl.kernel(
        _sc_body,
        out_type=(
            jax.ShapeDtypeStruct((BATCH * PAD,), jnp.float32),
            jax.ShapeDtypeStruct((BATCH * PAD,), jnp.int32),
            jax.ShapeDtypeStruct((BATCH * PAD * 4,), jnp.float32),
        ),
        mesh=plsc.VectorSubcoreMesh(core_axis_name="c", subcore_axis_name="s"),
        scratch_types=[
            pltpu.VMEM((N,), jnp.float32),
            pltpu.VMEM((4 * NQ,), jnp.float32),
            pltpu.VMEM((BATCH * 16,), jnp.float32),
            pltpu.VMEM((NB * 16,), jnp.int32),
            pltpu.VMEM((NB,), jnp.int32),
            pltpu.VMEM((CAP + 16,), jnp.int32),
            pltpu.VMEM((CAP + 16,), jnp.int32),
            pltpu.VMEM((PAD,), jnp.int32),
            pltpu.VMEM((PAD,), jnp.int32),
            pltpu.VMEM((PAD,), jnp.int32),
            pltpu.VMEM((PAD,), jnp.int32),
            pltpu.VMEM((PAD,), jnp.float32),
            pltpu.VMEM((PAD,), jnp.int32),
            pltpu.VMEM((PAD * 4,), jnp.float32),
        ],
    )


def kernel(pred_logits, pred_boxes, orig_target_sizes):
    batch, nq, cls = pred_logits.shape
    assert (batch, nq, cls) == (BATCH, NQ, CLS)
    scores = jax.nn.sigmoid(pred_logits).reshape(batch * nq * cls)
    boxes_flat = pred_boxes.reshape(batch * nq * 4)
    wh = orig_target_sizes.astype(jnp.float32)
    sizes16 = jnp.tile(wh, (1, 8)).reshape(batch * 16)
    o_s, o_l, o_b = _make_sc_call()(scores, boxes_flat, sizes16)
    scores_out = o_s.reshape(batch, PAD)[:, :K]
    labels_out = o_l.reshape(batch, PAD)[:, :K]
    boxes_out = o_b.reshape(batch, PAD, 4)[:, :K, :]
    return scores_out, labels_out, boxes_out


# speedup vs baseline: 8.2254x; 8.2254x over previous
import functools

import jax
import jax.numpy as jnp
from jax import lax
from jax.experimental import pallas as pl
from jax.experimental.pallas import tpu as pltpu
from jax.experimental.pallas import tpu_sc as plsc

CLS = 80
NQ = 1000
K = 300
N = NQ * CLS
NV = N // 16
NB = 1024
NBC = NB // 16
CAP = 8192
PAD = 320
BATCH = 64


def _scalar(x):
    return x[0] if getattr(x, "ndim", 0) else x


def _sc_body(scores_hbm, boxes_hbm, sizes_hbm, out_s_hbm, out_l_hbm, out_b_hbm,
             sc_v, bx_v, sz_v, hist_v, tot_v, cu_v, ci_v, fu_v, fi_v,
             uu_v, oi_v, ou_s, ol_v, ob_v):
    lane = lax.iota(jnp.int32, 16)
    ones = jnp.ones((16,), jnp.int32)
    zz = jnp.zeros((16,), jnp.int32)
    big = jnp.int32(1 << 30)
    wid = lax.axis_index("s") * 2 + lax.axis_index("c")
    pltpu.sync_copy(sizes_hbm, sz_v)

    def _zero_hist():
        def zb(i, _):
            hist_v[pl.ds(i * 16, 16)] = zz
            return 0
        lax.fori_loop(0, NB, zb, 0)

    def _rowsums():
        def rs_chunk(jc, _):
            acc = zz
            for l in range(16):
                row = hist_v[pl.ds((jc * 16 + l) * 16, 16)]
                acc = jnp.where(lane == l, jnp.sum(row), acc)
            tot_v[pl.ds(jc * 16, 16)] = acc
            return 0
        lax.fori_loop(0, NBC, rs_chunk, 0)

    def _boundary(tgt):
        def chunk(jj, carry):
            fb, fna, cum = carry
            jc = NBC - 1 - jj
            rs = tot_v[pl.ds(jc * 16, 16)]
            cs = plsc.cumsum(lax.rev(rs, (0,)))
            cond = (cum + cs) >= tgt
            t = _scalar(plsc.all_reduce_ffs(cond))
            hit = (t < 16) & (fb < 0)
            na_extra = jnp.sum(jnp.where(lane == t - 1, cs, 0))
            b_new = jc * 16 + 15 - t
            fb = jnp.where(hit, b_new, fb)
            fna = jnp.where(hit, cum + na_extra, fna)
            cum = cum + cs[15]
            return fb, fna, cum
        fb, fna, _ = lax.fori_loop(
            0, NBC, chunk, (jnp.int32(-1), jnp.int32(0), jnp.int32(0)))
        return fb, fna

    def per_image(ii, _):
        b = wid * 2 + ii
        pltpu.sync_copy(scores_hbm.at[pl.ds(b * N, N)], sc_v)
        pltpu.sync_copy(boxes_hbm.at[pl.ds(b * 4 * NQ, 4 * NQ)], bx_v)

        def load_full(i):
            u = jnp.maximum(sc_v[pl.ds(i * 16, 16)], 1)
            return u, i * 16 + lane, None

        def and_valid(cond, valid):
            return cond if valid is None else (cond & valid)

        _zero_hist()

        def h1(i, _):
            u, _idx, _val = load_full(i)
            d = u >> 20
            plsc.addupdate_scatter(hist_v, [d * 16 + lane], ones, mask=lane < 16)
            return 0
        lax.fori_loop(0, NV, h1, 0)
        _rowsums()
        b1, n_hi1 = _boundary(jnp.int32(K))
        row = tot_v[pl.ds((b1 // 16) * 16, 16)]
        cnt_b1 = jnp.sum(jnp.where(lane == (b1 % 16), row, 0))
        nc = n_hi1 + cnt_b1

        def refine(load_fn, nloops):
            _zero_hist()

            def h2(i, _):
                u, _idx, valid = load_fn(i)
                m = and_valid((u >> 20) == b1, valid)
                d = (u >> 10) & 0x3FF
                plsc.addupdate_scatter(hist_v, [d * 16 + lane], ones, mask=m)
                return 0
            lax.fori_loop(0, nloops, h2, 0)
            _rowsums()
            b2, n_hi2 = _boundary(K - n_hi1)
            pre2 = (b1 << 10) | b2
            _zero_hist()

            def h3(i, _):
                u, _idx, valid = load_fn(i)
                m = and_valid((u >> 10) == pre2, valid)
                d = u & 0x3FF
                plsc.addupdate_scatter(hist_v, [d * 16 + lane], ones, mask=m)
                return 0
            lax.fori_loop(0, nloops, h3, 0)
            _rowsums()
            b3, n_hi3 = _boundary(K - n_hi1 - n_hi2)
            thr = (pre2 << 10) | b3
            n_strict = n_hi1 + n_hi2 + n_hi3
            m_eq = K - n_strict

            def cps(i, off):
                u, idx, valid = load_fn(i)
                m = and_valid(u > thr, valid)
                plsc.store_compressed(fu_v.at[pl.ds(off, 16)], u, mask=m)
                plsc.store_compressed(fi_v.at[pl.ds(off, 16)], idx, mask=m)
                return off + _scalar(plsc.all_reduce_population_count(m))
            lax.fori_loop(0, nloops, cps, jnp.int32(0))

            def cpe(i, off):
                u, idx, valid = load_fn(i)
                m = and_valid(u == thr, valid)
                dst = n_strict + jnp.minimum(off, m_eq)
                plsc.store_compressed(fu_v.at[pl.ds(dst, 16)], u, mask=m)
                plsc.store_compressed(fi_v.at[pl.ds(dst, 16)], idx, mask=m)
                return off + _scalar(plsc.all_reduce_population_count(m))
            lax.fori_loop(0, nloops, cpe, jnp.int32(0))
            fu_v[pl.ds(K, 16)] = zz

        @pl.when(nc <= CAP)
        def _():
            def cp(i, off):
                u, idx, _val = load_full(i)
                m = (u >> 20) >= b1
                plsc.store_compressed(cu_v.at[pl.ds(off, 16)], u, mask=m)
                plsc.store_compressed(ci_v.at[pl.ds(off, 16)], idx, mask=m)
                return off + _scalar(plsc.all_reduce_population_count(m))
            lax.fori_loop(0, NV, cp, jnp.int32(0))

            def load_c(i):
                u = cu_v[pl.ds(i * 16, 16)]
                idx = ci_v[pl.ds(i * 16, 16)]
                return u, idx, (i * 16 + lane) < nc
            refine(load_c, (nc + 15) >> 4)

        @pl.when(nc > CAP)
        def _():
            refine(load_full, NV)

        def ext(e, ws):
            m_run = zz
            j_run = zz
            for j in range(19):
                upd = ws[j] > m_run
                j_run = jnp.where(upd, j, j_run)
                m_run = jnp.where(upd, ws[j], m_run)
            m = jnp.max(m_run)
            pv = jnp.where(m_run == m, j_run * 16 + lane, big)
            p = jnp.min(pv)
            fidxv = plsc.load_gather(fi_v, [zz + p])
            plsc.store_scatter(uu_v, [zz + e], zz + m, mask=lane == 0)
            plsc.store_scatter(oi_v, [zz + e], fidxv, mask=lane == 0)
            return tuple(
                jnp.where(j * 16 + lane == p, 0, ws[j]) for j in range(19))
        uu_v[pl.ds(K, 16)] = zz
        uu_v[pl.ds(PAD - 16, 16)] = zz
        oi_v[pl.ds(K, 16)] = zz
        oi_v[pl.ds(PAD - 16, 16)] = zz
        ws0 = tuple(fu_v[pl.ds(j * 16, 16)] for j in range(19))
        lax.fori_loop(0, K, ext, ws0)

        wv = plsc.load_gather(sz_v, [zz + b * 16])
        hv = plsc.load_gather(sz_v, [zz + b * 16 + 1])

        def post(j, _):
            ou_s[pl.ds(j * 16, 16)] = uu_v[pl.ds(j * 16, 16)]
            fidx = oi_v[pl.ds(j * 16, 16)]
            ol_v[pl.ds(j * 16, 16)] = fidx % CLS
            q4 = (fidx // CLS) * 4
            cx = plsc.load_gather(bx_v, [q4])
            cy = plsc.load_gather(bx_v, [q4 + 1])
            ww = plsc.load_gather(bx_v, [q4 + 2])
            hh = plsc.load_gather(bx_v, [q4 + 3])
            pos = (j * 16 + lane) * 4
            plsc.store_scatter(ob_v, [pos], (cx - 0.5 * ww) * wv)
            plsc.store_scatter(ob_v, [pos + 1], (cy - 0.5 * hh) * hv)
            plsc.store_scatter(ob_v, [pos + 2], (cx + 0.5 * ww) * wv)
            plsc.store_scatter(ob_v, [pos + 3], (cy + 0.5 * hh) * hv)
            return 0
        lax.fori_loop(0, PAD // 16, post, 0)

        pltpu.sync_copy(ou_s, out_s_hbm.at[pl.ds(b * PAD, PAD)])
        pltpu.sync_copy(ol_v, out_l_hbm.at[pl.ds(b * PAD, PAD)])
        pltpu.sync_copy(ob_v, out_b_hbm.at[pl.ds(b * PAD * 4, PAD * 4)])
        return 0

    lax.fori_loop(0, 2, per_image, 0)


@functools.cache
def _make_sc_call():
    return pl.kernel(
        _sc_body,
        out_type=(
            jax.ShapeDtypeStruct((BATCH * PAD,), jnp.int32),
            jax.ShapeDtypeStruct((BATCH * PAD,), jnp.int32),
            jax.ShapeDtypeStruct((BATCH * PAD * 4,), jnp.float32),
        ),
        mesh=plsc.VectorSubcoreMesh(core_axis_name="c", subcore_axis_name="s"),
        compiler_params=pltpu.CompilerParams(
            use_tc_tiling_on_sc=False, needs_layout_passes=False),
        scratch_types=[
            pltpu.VMEM((N,), jnp.int32),
            pltpu.VMEM((4 * NQ,), jnp.float32),
            pltpu.VMEM((BATCH * 16,), jnp.float32),
            pltpu.VMEM((NB * 16,), jnp.int32),
            pltpu.VMEM((NB,), jnp.int32),
            pltpu.VMEM((CAP + 16,), jnp.int32),
            pltpu.VMEM((CAP + 16,), jnp.int32),
            pltpu.VMEM((PAD,), jnp.int32),
            pltpu.VMEM((PAD,), jnp.int32),
            pltpu.VMEM((PAD,), jnp.int32),
            pltpu.VMEM((PAD,), jnp.int32),
            pltpu.VMEM((PAD,), jnp.int32),
            pltpu.VMEM((PAD,), jnp.int32),
            pltpu.VMEM((PAD * 4,), jnp.float32),
        ],
    )


def kernel(pred_logits, pred_boxes, orig_target_sizes):
    batch, nq, cls = pred_logits.shape
    assert (batch, nq, cls) == (BATCH, NQ, CLS)
    scores = jax.nn.sigmoid(pred_logits).reshape(batch * nq * cls)
    score_bits = lax.bitcast_convert_type(scores, jnp.int32)
    boxes_flat = pred_boxes.reshape(batch * nq * 4)
    wh = orig_target_sizes.astype(jnp.float32)
    sizes16 = jnp.tile(wh, (1, 8)).reshape(batch * 16)
    o_s, o_l, o_b = _make_sc_call()(score_bits, boxes_flat, sizes16)
    scores_out = lax.bitcast_convert_type(o_s, jnp.float32).reshape(batch, PAD)[:, :K]
    labels_out = o_l.reshape(batch, PAD)[:, :K]
    boxes_out = o_b.reshape(batch, PAD, 4)[:, :K, :]
    return scores_out, labels_out, boxes_out


# speedup vs baseline: 9.0582x; 1.1013x over previous
import functools

import jax
import jax.numpy as jnp
from jax import lax
from jax.experimental import pallas as pl
from jax.experimental.pallas import tpu as pltpu
from jax.experimental.pallas import tpu_sc as plsc

CLS = 80
NQ = 1000
K = 300
N = NQ * CLS
NV = N // 16
NB = 1024
NBC = NB // 16
CAP = 8192
PAD = 320
BATCH = 64


def _scalar(x):
    return x[0] if getattr(x, "ndim", 0) else x


def _sc_body(scores_hbm, boxes_hbm, sizes_hbm, out_s_hbm, out_l_hbm, out_b_hbm,
             sc_v, bx_v, sz_v, tot_v, cu_v, ci_v, fu_v, fi_v,
             uu_v, oi_v, ou_s, ol_v, ob_v):
    lane = lax.iota(jnp.int32, 16)
    ones = jnp.ones((16,), jnp.int32)
    zz = jnp.zeros((16,), jnp.int32)
    big = jnp.int32(1 << 30)
    wid = lax.axis_index("s") * 2 + lax.axis_index("c")
    pltpu.sync_copy(sizes_hbm, sz_v)

    def _zero_hist():
        def zb(i, _):
            tot_v[pl.ds(i * 16, 16)] = zz
            return 0
        lax.fori_loop(0, NBC, zb, 0)

    def _boundary(tgt):
        def chunk(jj, carry):
            fb, fna, cum = carry
            jc = NBC - 1 - jj
            rs = tot_v[pl.ds(jc * 16, 16)]
            cs = plsc.cumsum(lax.rev(rs, (0,)))
            cond = (cum + cs) >= tgt
            t = _scalar(plsc.all_reduce_ffs(cond))
            hit = (t < 16) & (fb < 0)
            na_extra = jnp.sum(jnp.where(lane == t - 1, cs, 0))
            b_new = jc * 16 + 15 - t
            fb = jnp.where(hit, b_new, fb)
            fna = jnp.where(hit, cum + na_extra, fna)
            cum = cum + cs[15]
            return fb, fna, cum
        fb, fna, _ = lax.fori_loop(
            0, NBC, chunk, (jnp.int32(-1), jnp.int32(0), jnp.int32(0)))
        return fb, fna

    def per_image(ii, _):
        b = wid * 2 + ii
        pltpu.sync_copy(scores_hbm.at[pl.ds(b * N, N)], sc_v)
        pltpu.sync_copy(boxes_hbm.at[pl.ds(b * 4 * NQ, 4 * NQ)], bx_v)

        def load_full(i):
            u = jnp.maximum(sc_v[pl.ds(i * 16, 16)], 1)
            return u, i * 16 + lane, None

        def and_valid(cond, valid):
            return cond if valid is None else (cond & valid)

        _zero_hist()

        def h1(i, _):
            u, _idx, _val = load_full(i)
            d = u >> 20
            plsc.addupdate_scatter(tot_v, [d], ones, mask=lane < 16)
            return 0
        lax.fori_loop(0, NV, h1, 0)
        b1, n_hi1 = _boundary(jnp.int32(K))
        row = tot_v[pl.ds((b1 // 16) * 16, 16)]
        cnt_b1 = jnp.sum(jnp.where(lane == (b1 % 16), row, 0))
        nc = n_hi1 + cnt_b1

        def refine(load_fn, nloops):
            _zero_hist()

            def h2(i, _):
                u, _idx, valid = load_fn(i)
                m = and_valid((u >> 20) == b1, valid)
                d = (u >> 10) & 0x3FF
                plsc.addupdate_scatter(tot_v, [d], ones, mask=m)
                return 0
            lax.fori_loop(0, nloops, h2, 0)
            b2, n_hi2 = _boundary(K - n_hi1)
            pre2 = (b1 << 10) | b2
            _zero_hist()

            def h3(i, _):
                u, _idx, valid = load_fn(i)
                m = and_valid((u >> 10) == pre2, valid)
                d = u & 0x3FF
                plsc.addupdate_scatter(tot_v, [d], ones, mask=m)
                return 0
            lax.fori_loop(0, nloops, h3, 0)
            b3, n_hi3 = _boundary(K - n_hi1 - n_hi2)
            thr = (pre2 << 10) | b3
            n_strict = n_hi1 + n_hi2 + n_hi3
            m_eq = K - n_strict

            def cps(i, off):
                u, idx, valid = load_fn(i)
                m = and_valid(u > thr, valid)
                plsc.store_compressed(fu_v.at[pl.ds(off, 16)], u, mask=m)
                plsc.store_compressed(fi_v.at[pl.ds(off, 16)], idx, mask=m)
                return off + _scalar(plsc.all_reduce_population_count(m))
            lax.fori_loop(0, nloops, cps, jnp.int32(0))

            def cpe(i, off):
                u, idx, valid = load_fn(i)
                m = and_valid(u == thr, valid)
                dst = n_strict + jnp.minimum(off, m_eq)
                plsc.store_compressed(fu_v.at[pl.ds(dst, 16)], u, mask=m)
                plsc.store_compressed(fi_v.at[pl.ds(dst, 16)], idx, mask=m)
                return off + _scalar(plsc.all_reduce_population_count(m))
            lax.fori_loop(0, nloops, cpe, jnp.int32(0))
            fu_v[pl.ds(K, 16)] = zz

        @pl.when(nc <= CAP)
        def _():
            def cp(i, off):
                u, idx, _val = load_full(i)
                m = (u >> 20) >= b1
                plsc.store_compressed(cu_v.at[pl.ds(off, 16)], u, mask=m)
                plsc.store_compressed(ci_v.at[pl.ds(off, 16)], idx, mask=m)
                return off + _scalar(plsc.all_reduce_population_count(m))
            lax.fori_loop(0, NV, cp, jnp.int32(0))

            def load_c(i):
                u = cu_v[pl.ds(i * 16, 16)]
                idx = ci_v[pl.ds(i * 16, 16)]
                return u, idx, (i * 16 + lane) < nc
            refine(load_c, (nc + 15) >> 4)

        @pl.when(nc > CAP)
        def _():
            refine(load_full, NV)

        def ext(e, ws):
            m_run = zz
            j_run = zz
            for j in range(19):
                upd = ws[j] > m_run
                j_run = jnp.where(upd, j, j_run)
                m_run = jnp.where(upd, ws[j], m_run)
            m = jnp.max(m_run)
            pv = jnp.where(m_run == m, j_run * 16 + lane, big)
            p = jnp.min(pv)
            fidxv = plsc.load_gather(fi_v, [zz + p])
            plsc.store_scatter(uu_v, [zz + e], zz + m, mask=lane == 0)
            plsc.store_scatter(oi_v, [zz + e], fidxv, mask=lane == 0)
            return tuple(
                jnp.where(j * 16 + lane == p, 0, ws[j]) for j in range(19))
        uu_v[pl.ds(K, 16)] = zz
        uu_v[pl.ds(PAD - 16, 16)] = zz
        oi_v[pl.ds(K, 16)] = zz
        oi_v[pl.ds(PAD - 16, 16)] = zz
        ws0 = tuple(fu_v[pl.ds(j * 16, 16)] for j in range(19))
        lax.fori_loop(0, K, ext, ws0)

        wv = plsc.load_gather(sz_v, [zz + b * 16])
        hv = plsc.load_gather(sz_v, [zz + b * 16 + 1])

        def post(j, _):
            ou_s[pl.ds(j * 16, 16)] = uu_v[pl.ds(j * 16, 16)]
            fidx = oi_v[pl.ds(j * 16, 16)]
            ol_v[pl.ds(j * 16, 16)] = fidx % CLS
            q4 = (fidx // CLS) * 4
            cx = plsc.load_gather(bx_v, [q4])
            cy = plsc.load_gather(bx_v, [q4 + 1])
            ww = plsc.load_gather(bx_v, [q4 + 2])
            hh = plsc.load_gather(bx_v, [q4 + 3])
            pos = (j * 16 + lane) * 4
            plsc.store_scatter(ob_v, [pos], (cx - 0.5 * ww) * wv)
            plsc.store_scatter(ob_v, [pos + 1], (cy - 0.5 * hh) * hv)
            plsc.store_scatter(ob_v, [pos + 2], (cx + 0.5 * ww) * wv)
            plsc.store_scatter(ob_v, [pos + 3], (cy + 0.5 * hh) * hv)
            return 0
        lax.fori_loop(0, PAD // 16, post, 0)

        pltpu.sync_copy(ou_s, out_s_hbm.at[pl.ds(b * PAD, PAD)])
        pltpu.sync_copy(ol_v, out_l_hbm.at[pl.ds(b * PAD, PAD)])
        pltpu.sync_copy(ob_v, out_b_hbm.at[pl.ds(b * PAD * 4, PAD * 4)])
        return 0

    lax.fori_loop(0, 2, per_image, 0)


@functools.cache
def _make_sc_call():
    return pl.kernel(
        _sc_body,
        out_type=(
            jax.ShapeDtypeStruct((BATCH * PAD,), jnp.int32),
            jax.ShapeDtypeStruct((BATCH * PAD,), jnp.int32),
            jax.ShapeDtypeStruct((BATCH * PAD * 4,), jnp.float32),
        ),
        mesh=plsc.VectorSubcoreMesh(core_axis_name="c", subcore_axis_name="s"),
        compiler_params=pltpu.CompilerParams(
            use_tc_tiling_on_sc=False, needs_layout_passes=False),
        scratch_types=[
            pltpu.VMEM((N,), jnp.int32),
            pltpu.VMEM((4 * NQ,), jnp.float32),
            pltpu.VMEM((BATCH * 16,), jnp.float32),
            pltpu.VMEM((NB,), jnp.int32),
            pltpu.VMEM((CAP + 16,), jnp.int32),
            pltpu.VMEM((CAP + 16,), jnp.int32),
            pltpu.VMEM((PAD,), jnp.int32),
            pltpu.VMEM((PAD,), jnp.int32),
            pltpu.VMEM((PAD,), jnp.int32),
            pltpu.VMEM((PAD,), jnp.int32),
            pltpu.VMEM((PAD,), jnp.int32),
            pltpu.VMEM((PAD,), jnp.int32),
            pltpu.VMEM((PAD * 4,), jnp.float32),
        ],
    )


def kernel(pred_logits, pred_boxes, orig_target_sizes):
    batch, nq, cls = pred_logits.shape
    assert (batch, nq, cls) == (BATCH, NQ, CLS)
    scores = jax.nn.sigmoid(pred_logits).reshape(batch * nq * cls)
    score_bits = lax.bitcast_convert_type(scores, jnp.int32)
    boxes_flat = pred_boxes.reshape(batch * nq * 4)
    wh = orig_target_sizes.astype(jnp.float32)
    sizes16 = jnp.tile(wh, (1, 8)).reshape(batch * 16)
    o_s, o_l, o_b = _make_sc_call()(score_bits, boxes_flat, sizes16)
    scores_out = lax.bitcast_convert_type(o_s, jnp.float32).reshape(batch, PAD)[:, :K]
    labels_out = o_l.reshape(batch, PAD)[:, :K]
    boxes_out = o_b.reshape(batch, PAD, 4)[:, :K, :]
    return scores_out, labels_out, boxes_out


# speedup vs baseline: 9.5912x; 1.0588x over previous
import functools

import jax
import jax.numpy as jnp
from jax import lax
from jax.experimental import pallas as pl
from jax.experimental.pallas import tpu as pltpu
from jax.experimental.pallas import tpu_sc as plsc

CLS = 80
NQ = 1000
K = 300
N = NQ * CLS
NV = N // 16
NB = 1024
NBC = NB // 16
CAP = 8192
PAD = 320
BATCH = 64


def _scalar(x):
    return x[0] if getattr(x, "ndim", 0) else x


def _sc_body(scores_hbm, boxes_hbm, sizes_hbm, out_s_hbm, out_l_hbm, out_b_hbm,
             sc_v, bx_v, sz_v, tot_v, cu_v, ci_v, fu_v, fi_v,
             uu_v, oi_v, ou_s, ol_v, ob_v):
    lane = lax.iota(jnp.int32, 16)
    ones = jnp.ones((16,), jnp.int32)
    zz = jnp.zeros((16,), jnp.int32)
    big = jnp.int32(1 << 30)
    wid = lax.axis_index("s") * 2 + lax.axis_index("c")
    pltpu.sync_copy(sizes_hbm, sz_v)

    def _zero_hist():
        def zb(i, _):
            tot_v[pl.ds(i * 16, 16)] = zz
            return 0
        lax.fori_loop(0, NBC, zb, 0)

    def _boundary(tgt):
        def chunk(jj, carry):
            fb, fna, cum = carry
            jc = NBC - 1 - jj
            rs = tot_v[pl.ds(jc * 16, 16)]
            cs = plsc.cumsum(lax.rev(rs, (0,)))
            cond = (cum + cs) >= tgt
            t = _scalar(plsc.all_reduce_ffs(cond))
            hit = (t < 16) & (fb < 0)
            na_extra = jnp.sum(jnp.where(lane == t - 1, cs, 0))
            b_new = jc * 16 + 15 - t
            fb = jnp.where(hit, b_new, fb)
            fna = jnp.where(hit, cum + na_extra, fna)
            cum = cum + cs[15]
            return fb, fna, cum
        fb, fna, _ = lax.fori_loop(
            0, NBC, chunk, (jnp.int32(-1), jnp.int32(0), jnp.int32(0)))
        return fb, fna

    def per_image(ii, _):
        b = wid * 2 + ii
        pltpu.sync_copy(scores_hbm.at[b], sc_v)
        pltpu.sync_copy(boxes_hbm.at[b], bx_v)

        def load_full(i):
            u = jnp.maximum(sc_v[pl.ds(i * 16, 16)], 1)
            return u, i * 16 + lane, None

        def and_valid(cond, valid):
            return cond if valid is None else (cond & valid)

        _zero_hist()

        def h1(i, _):
            u, _idx, _val = load_full(i)
            d = u >> 20
            plsc.addupdate_scatter(tot_v, [d], ones, mask=lane < 16)
            return 0
        lax.fori_loop(0, NV, h1, 0)
        b1, n_hi1 = _boundary(jnp.int32(K))
        row = tot_v[pl.ds((b1 // 16) * 16, 16)]
        cnt_b1 = jnp.sum(jnp.where(lane == (b1 % 16), row, 0))
        nc = n_hi1 + cnt_b1

        def refine(load_fn, nloops):
            _zero_hist()

            def h2(i, _):
                u, _idx, valid = load_fn(i)
                m = and_valid((u >> 20) == b1, valid)
                d = (u >> 10) & 0x3FF
                plsc.addupdate_scatter(tot_v, [d], ones, mask=m)
                return 0
            lax.fori_loop(0, nloops, h2, 0)
            b2, n_hi2 = _boundary(K - n_hi1)
            pre2 = (b1 << 10) | b2
            _zero_hist()

            def h3(i, _):
                u, _idx, valid = load_fn(i)
                m = and_valid((u >> 10) == pre2, valid)
                d = u & 0x3FF
                plsc.addupdate_scatter(tot_v, [d], ones, mask=m)
                return 0
            lax.fori_loop(0, nloops, h3, 0)
            b3, n_hi3 = _boundary(K - n_hi1 - n_hi2)
            thr = (pre2 << 10) | b3
            n_strict = n_hi1 + n_hi2 + n_hi3
            m_eq = K - n_strict

            def cps(i, off):
                u, idx, valid = load_fn(i)
                m = and_valid(u > thr, valid)
                plsc.store_compressed(fu_v.at[pl.ds(off, 16)], u, mask=m)
                plsc.store_compressed(fi_v.at[pl.ds(off, 16)], idx, mask=m)
                return off + _scalar(plsc.all_reduce_population_count(m))
            lax.fori_loop(0, nloops, cps, jnp.int32(0))

            def cpe(i, off):
                u, idx, valid = load_fn(i)
                m = and_valid(u == thr, valid)
                dst = n_strict + jnp.minimum(off, m_eq)
                plsc.store_compressed(fu_v.at[pl.ds(dst, 16)], u, mask=m)
                plsc.store_compressed(fi_v.at[pl.ds(dst, 16)], idx, mask=m)
                return off + _scalar(plsc.all_reduce_population_count(m))
            lax.fori_loop(0, nloops, cpe, jnp.int32(0))
            fu_v[pl.ds(K, 16)] = zz

        @pl.when(nc <= CAP)
        def _():
            def cp(i, off):
                u, idx, _val = load_full(i)
                m = (u >> 20) >= b1
                plsc.store_compressed(cu_v.at[pl.ds(off, 16)], u, mask=m)
                plsc.store_compressed(ci_v.at[pl.ds(off, 16)], idx, mask=m)
                return off + _scalar(plsc.all_reduce_population_count(m))
            lax.fori_loop(0, NV, cp, jnp.int32(0))

            def load_c(i):
                u = cu_v[pl.ds(i * 16, 16)]
                idx = ci_v[pl.ds(i * 16, 16)]
                return u, idx, (i * 16 + lane) < nc
            refine(load_c, (nc + 15) >> 4)

        @pl.when(nc > CAP)
        def _():
            refine(load_full, NV)

        def ext(e, ws):
            m_run = zz
            j_run = zz
            for j in range(19):
                upd = ws[j] > m_run
                j_run = jnp.where(upd, j, j_run)
                m_run = jnp.where(upd, ws[j], m_run)
            m = jnp.max(m_run)
            pv = jnp.where(m_run == m, j_run * 16 + lane, big)
            p = jnp.min(pv)
            fidxv = plsc.load_gather(fi_v, [zz + p])
            plsc.store_scatter(uu_v, [zz + e], zz + m, mask=lane == 0)
            plsc.store_scatter(oi_v, [zz + e], fidxv, mask=lane == 0)
            return tuple(
                jnp.where(j * 16 + lane == p, 0, ws[j]) for j in range(19))
        uu_v[pl.ds(K, 16)] = zz
        uu_v[pl.ds(PAD - 16, 16)] = zz
        oi_v[pl.ds(K, 16)] = zz
        oi_v[pl.ds(PAD - 16, 16)] = zz
        ws0 = tuple(fu_v[pl.ds(j * 16, 16)] for j in range(19))
        lax.fori_loop(0, K, ext, ws0)

        wv = plsc.load_gather(sz_v, [zz + b * 16])
        hv = plsc.load_gather(sz_v, [zz + b * 16 + 1])

        def post(j, _):
            ou_s[pl.ds(j * 16, 16)] = uu_v[pl.ds(j * 16, 16)]
            fidx = oi_v[pl.ds(j * 16, 16)]
            ol_v[pl.ds(j * 16, 16)] = fidx % CLS
            q4 = (fidx // CLS) * 4
            cx = plsc.load_gather(bx_v, [q4])
            cy = plsc.load_gather(bx_v, [q4 + 1])
            ww = plsc.load_gather(bx_v, [q4 + 2])
            hh = plsc.load_gather(bx_v, [q4 + 3])
            pos = (j * 16 + lane) * 4
            plsc.store_scatter(ob_v, [pos], (cx - 0.5 * ww) * wv)
            plsc.store_scatter(ob_v, [pos + 1], (cy - 0.5 * hh) * hv)
            plsc.store_scatter(ob_v, [pos + 2], (cx + 0.5 * ww) * wv)
            plsc.store_scatter(ob_v, [pos + 3], (cy + 0.5 * hh) * hv)
            return 0
        lax.fori_loop(0, PAD // 16, post, 0)

        pltpu.sync_copy(ou_s, out_s_hbm.at[b])
        pltpu.sync_copy(ol_v, out_l_hbm.at[b])
        pltpu.sync_copy(ob_v, out_b_hbm.at[b])
        return 0

    lax.fori_loop(0, 2, per_image, 0)


@functools.cache
def _make_sc_call():
    return pl.kernel(
        _sc_body,
        out_type=(
            jax.ShapeDtypeStruct((BATCH, PAD), jnp.int32),
            jax.ShapeDtypeStruct((BATCH, PAD), jnp.int32),
            jax.ShapeDtypeStruct((BATCH, PAD * 4), jnp.float32),
        ),
        mesh=plsc.VectorSubcoreMesh(core_axis_name="c", subcore_axis_name="s"),
        compiler_params=pltpu.CompilerParams(
            use_tc_tiling_on_sc=False, needs_layout_passes=False),
        scratch_types=[
            pltpu.VMEM((N,), jnp.int32),
            pltpu.VMEM((4 * NQ,), jnp.float32),
            pltpu.VMEM((BATCH * 16,), jnp.float32),
            pltpu.VMEM((NB,), jnp.int32),
            pltpu.VMEM((CAP + 16,), jnp.int32),
            pltpu.VMEM((CAP + 16,), jnp.int32),
            pltpu.VMEM((PAD,), jnp.int32),
            pltpu.VMEM((PAD,), jnp.int32),
            pltpu.VMEM((PAD,), jnp.int32),
            pltpu.VMEM((PAD,), jnp.int32),
            pltpu.VMEM((PAD,), jnp.int32),
            pltpu.VMEM((PAD,), jnp.int32),
            pltpu.VMEM((PAD * 4,), jnp.float32),
        ],
    )


def kernel(pred_logits, pred_boxes, orig_target_sizes):
    batch, nq, cls = pred_logits.shape
    assert (batch, nq, cls) == (BATCH, NQ, CLS)
    scores = jax.nn.sigmoid(pred_logits).reshape(batch, nq * cls)
    score_bits = lax.bitcast_convert_type(scores, jnp.int32)
    boxes_2d = pred_boxes.reshape(batch, nq * 4)
    wh = orig_target_sizes.astype(jnp.float32)
    sizes16 = jnp.tile(wh, (1, 8)).reshape(batch * 16)
    o_s, o_l, o_b = _make_sc_call()(score_bits, boxes_2d, sizes16)
    scores_out = lax.bitcast_convert_type(o_s, jnp.float32)[:, :K]
    labels_out = o_l[:, :K]
    boxes_out = o_b.reshape(batch, PAD, 4)[:, :K, :]
    return scores_out, labels_out, boxes_out


# speedup vs baseline: 9.8188x; 1.0237x over previous
import functools

import jax
import jax.numpy as jnp
from jax import lax
from jax.experimental import pallas as pl
from jax.experimental.pallas import tpu as pltpu
from jax.experimental.pallas import tpu_sc as plsc

CLS = 80
NQ = 1000
K = 300
N = NQ * CLS
NV = N // 16
NB = 1024
NBC = NB // 16
CAP = 8192
PAD = 320
BATCH = 64


def _scalar(x):
    return x[0] if getattr(x, "ndim", 0) else x


def _sc_body(scores_hbm, boxes_hbm, sizes_hbm, out_s_hbm, out_l_hbm, out_b_hbm,
             sc_v, bx_v, sz_v, tot_v, cu_v, ci_v, fu_v, fi_v,
             uu_v, oi_v, ou_s, ol_v, ob_v):
    lane = lax.iota(jnp.int32, 16)
    ones = jnp.ones((16,), jnp.int32)
    zz = jnp.zeros((16,), jnp.int32)
    big = jnp.int32(1 << 30)
    wid = lax.axis_index("s") * 2 + lax.axis_index("c")
    pltpu.sync_copy(sizes_hbm, sz_v)

    def _zero_hist():
        def zb(i, _):
            tot_v[pl.ds(i * 16, 16)] = zz
            return 0
        lax.fori_loop(0, NBC, zb, 0)

    def _boundary(tgt):
        def chunk(jj, carry):
            fb, fna, cum = carry
            jc = NBC - 1 - jj
            rs = tot_v[pl.ds(jc * 16, 16)]
            cs = plsc.cumsum(lax.rev(rs, (0,)))
            cond = (cum + cs) >= tgt
            t = _scalar(plsc.all_reduce_ffs(cond))
            hit = (t < 16) & (fb < 0)
            na_extra = jnp.sum(jnp.where(lane == t - 1, cs, 0))
            b_new = jc * 16 + 15 - t
            fb = jnp.where(hit, b_new, fb)
            fna = jnp.where(hit, cum + na_extra, fna)
            cum = cum + cs[15]
            return fb, fna, cum
        fb, fna, _ = lax.fori_loop(
            0, NBC, chunk, (jnp.int32(-1), jnp.int32(0), jnp.int32(0)))
        return fb, fna

    def per_image(ii, _):
        b = wid * 2 + ii
        pltpu.sync_copy(scores_hbm.at[b], sc_v)
        pltpu.sync_copy(boxes_hbm.at[b], bx_v)

        def load_full(i):
            u = jnp.maximum(sc_v[pl.ds(i * 16, 16)], 1)
            return u, i * 16 + lane, None

        def and_valid(cond, valid):
            return cond if valid is None else (cond & valid)

        _zero_hist()

        def h1(i, _):
            for t in range(4):
                u, _idx, _val = load_full(i * 4 + t)
                d = u >> 20
                plsc.addupdate_scatter(tot_v, [d], ones, mask=lane < 16)
            return 0
        lax.fori_loop(0, NV // 4, h1, 0)
        b1, n_hi1 = _boundary(jnp.int32(K))
        row = tot_v[pl.ds((b1 // 16) * 16, 16)]
        cnt_b1 = jnp.sum(jnp.where(lane == (b1 % 16), row, 0))
        nc = n_hi1 + cnt_b1

        def refine(load_fn, nloops):
            _zero_hist()

            def h2(i, _):
                u, _idx, valid = load_fn(i)
                m = and_valid((u >> 20) == b1, valid)
                d = (u >> 10) & 0x3FF
                plsc.addupdate_scatter(tot_v, [d], ones, mask=m)
                return 0
            lax.fori_loop(0, nloops, h2, 0)
            b2, n_hi2 = _boundary(K - n_hi1)
            pre2 = (b1 << 10) | b2
            _zero_hist()

            def h3(i, _):
                u, _idx, valid = load_fn(i)
                m = and_valid((u >> 10) == pre2, valid)
                d = u & 0x3FF
                plsc.addupdate_scatter(tot_v, [d], ones, mask=m)
                return 0
            lax.fori_loop(0, nloops, h3, 0)
            b3, n_hi3 = _boundary(K - n_hi1 - n_hi2)
            thr = (pre2 << 10) | b3
            n_strict = n_hi1 + n_hi2 + n_hi3
            m_eq = K - n_strict

            def cps(i, off):
                u, idx, valid = load_fn(i)
                m = and_valid(u > thr, valid)
                plsc.store_compressed(fu_v.at[pl.ds(off, 16)], u, mask=m)
                plsc.store_compressed(fi_v.at[pl.ds(off, 16)], idx, mask=m)
                return off + _scalar(plsc.all_reduce_population_count(m))
            lax.fori_loop(0, nloops, cps, jnp.int32(0))

            def cpe(i, off):
                u, idx, valid = load_fn(i)
                m = and_valid(u == thr, valid)
                dst = n_strict + jnp.minimum(off, m_eq)
                plsc.store_compressed(fu_v.at[pl.ds(dst, 16)], u, mask=m)
                plsc.store_compressed(fi_v.at[pl.ds(dst, 16)], idx, mask=m)
                return off + _scalar(plsc.all_reduce_population_count(m))
            lax.fori_loop(0, nloops, cpe, jnp.int32(0))
            fu_v[pl.ds(K, 16)] = zz

        @pl.when(nc <= CAP)
        def _():
            def cp(i, off):
                for t in range(4):
                    u, idx, _val = load_full(i * 4 + t)
                    m = (u >> 20) >= b1
                    plsc.store_compressed(cu_v.at[pl.ds(off, 16)], u, mask=m)
                    plsc.store_compressed(ci_v.at[pl.ds(off, 16)], idx, mask=m)
                    off = off + _scalar(plsc.all_reduce_population_count(m))
                return off
            lax.fori_loop(0, NV // 4, cp, jnp.int32(0))

            def load_c(i):
                u = cu_v[pl.ds(i * 16, 16)]
                idx = ci_v[pl.ds(i * 16, 16)]
                return u, idx, (i * 16 + lane) < nc
            refine(load_c, (nc + 15) >> 4)

        @pl.when(nc > CAP)
        def _():
            refine(load_full, NV)

        def ext(e, ws):
            nodes = [(ws[j], j * 16 + lane) for j in range(19)]
            while len(nodes) > 1:
                nxt = []
                for t in range(0, len(nodes) - 1, 2):
                    (ma, pa), (mb, pb) = nodes[t], nodes[t + 1]
                    ge = ma >= mb
                    nxt.append((jnp.where(ge, ma, mb), jnp.where(ge, pa, pb)))
                if len(nodes) % 2:
                    nxt.append(nodes[-1])
                nodes = nxt
            m_run, p_run = nodes[0]
            m = jnp.max(m_run)
            pv = jnp.where(m_run == m, p_run, big)
            p = jnp.min(pv)
            fidxv = plsc.load_gather(fi_v, [zz + p])
            plsc.store_scatter(uu_v, [zz + e], zz + m, mask=lane == 0)
            plsc.store_scatter(oi_v, [zz + e], fidxv, mask=lane == 0)
            return tuple(
                jnp.where(j * 16 + lane == p, 0, ws[j]) for j in range(19))
        uu_v[pl.ds(K, 16)] = zz
        uu_v[pl.ds(PAD - 16, 16)] = zz
        oi_v[pl.ds(K, 16)] = zz
        oi_v[pl.ds(PAD - 16, 16)] = zz
        ws0 = tuple(fu_v[pl.ds(j * 16, 16)] for j in range(19))
        lax.fori_loop(0, K, ext, ws0)

        wv = plsc.load_gather(sz_v, [zz + b * 16])
        hv = plsc.load_gather(sz_v, [zz + b * 16 + 1])

        def post(j, _):
            ou_s[pl.ds(j * 16, 16)] = uu_v[pl.ds(j * 16, 16)]
            fidx = oi_v[pl.ds(j * 16, 16)]
            ol_v[pl.ds(j * 16, 16)] = fidx % CLS
            q4 = (fidx // CLS) * 4
            cx = plsc.load_gather(bx_v, [q4])
            cy = plsc.load_gather(bx_v, [q4 + 1])
            ww = plsc.load_gather(bx_v, [q4 + 2])
            hh = plsc.load_gather(bx_v, [q4 + 3])
            pos = (j * 16 + lane) * 4
            plsc.store_scatter(ob_v, [pos], (cx - 0.5 * ww) * wv)
            plsc.store_scatter(ob_v, [pos + 1], (cy - 0.5 * hh) * hv)
            plsc.store_scatter(ob_v, [pos + 2], (cx + 0.5 * ww) * wv)
            plsc.store_scatter(ob_v, [pos + 3], (cy + 0.5 * hh) * hv)
            return 0
        lax.fori_loop(0, PAD // 16, post, 0)

        pltpu.sync_copy(ou_s, out_s_hbm.at[b])
        pltpu.sync_copy(ol_v, out_l_hbm.at[b])
        pltpu.sync_copy(ob_v, out_b_hbm.at[b])
        return 0

    lax.fori_loop(0, 2, per_image, 0)


@functools.cache
def _make_sc_call():
    return pl.kernel(
        _sc_body,
        out_type=(
            jax.ShapeDtypeStruct((BATCH, PAD), jnp.int32),
            jax.ShapeDtypeStruct((BATCH, PAD), jnp.int32),
            jax.ShapeDtypeStruct((BATCH, PAD * 4), jnp.float32),
        ),
        mesh=plsc.VectorSubcoreMesh(core_axis_name="c", subcore_axis_name="s"),
        compiler_params=pltpu.CompilerParams(
            use_tc_tiling_on_sc=False, needs_layout_passes=False),
        scratch_types=[
            pltpu.VMEM((N,), jnp.int32),
            pltpu.VMEM((4 * NQ,), jnp.float32),
            pltpu.VMEM((BATCH * 16,), jnp.float32),
            pltpu.VMEM((NB,), jnp.int32),
            pltpu.VMEM((CAP + 16,), jnp.int32),
            pltpu.VMEM((CAP + 16,), jnp.int32),
            pltpu.VMEM((PAD,), jnp.int32),
            pltpu.VMEM((PAD,), jnp.int32),
            pltpu.VMEM((PAD,), jnp.int32),
            pltpu.VMEM((PAD,), jnp.int32),
            pltpu.VMEM((PAD,), jnp.int32),
            pltpu.VMEM((PAD,), jnp.int32),
            pltpu.VMEM((PAD * 4,), jnp.float32),
        ],
    )


def kernel(pred_logits, pred_boxes, orig_target_sizes):
    batch, nq, cls = pred_logits.shape
    assert (batch, nq, cls) == (BATCH, NQ, CLS)
    scores = jax.nn.sigmoid(pred_logits).reshape(batch, nq * cls)
    score_bits = lax.bitcast_convert_type(scores, jnp.int32)
    boxes_2d = pred_boxes.reshape(batch, nq * 4)
    wh = orig_target_sizes.astype(jnp.float32)
    sizes16 = jnp.tile(wh, (1, 8)).reshape(batch * 16)
    o_s, o_l, o_b = _make_sc_call()(score_bits, boxes_2d, sizes16)
    scores_out = lax.bitcast_convert_type(o_s, jnp.float32)[:, :K]
    labels_out = o_l[:, :K]
    boxes_out = o_b.reshape(batch, PAD, 4)[:, :K, :]
    return scores_out, labels_out, boxes_out
